# Initial kernel scaffold; baseline (speedup 1.0000x reference)
import jax
import jax.numpy as jnp
from jax.experimental import pallas as pl


def kernel(x_0, incidence_1, W):
    raise NotImplementedError("write your pallas kernel here")


# speedup vs baseline: 1.7079x; 1.7079x over previous
import jax
import jax.numpy as jnp
from jax.experimental import pallas as pl
from jax.experimental.pallas import tpu as pltpu

_ALPHA = 0.1
_BETA = 0.5
_NB = 400


def _pass1(x_ref, inc_ref, m01T_ref, colsum_ref, wsum_ref):
    i = pl.program_id(0)
    blk = inc_ref[...]
    xT = jnp.transpose(x_ref[...])
    m = jax.lax.dot_general(xT, blk, (((1,), (0,)), ((), ())),
                            preferred_element_type=jnp.float32)
    nd = jnp.sum(blk, axis=1, keepdims=True)
    cs = jnp.sum(blk, axis=0, keepdims=True)
    ws = jnp.sum(blk * nd, axis=0, keepdims=True)

    @pl.when(i == 0)
    def _init():
        m01T_ref[...] = m
        colsum_ref[...] = cs
        wsum_ref[...] = ws

    @pl.when(i != 0)
    def _acc():
        m01T_ref[...] += m
        colsum_ref[...] += cs
        wsum_ref[...] += ws


def _pass2(inc_ref, x_ref, m01T_ref, colsum_ref, wsum_ref, wT_ref,
           out_ref, msc_ref):
    i = pl.program_id(0)

    @pl.when(i == 0)
    def _prep():
        inv_e = jax.lax.rsqrt(wsum_ref[...] / colsum_ref[...])
        msc_ref[...] = m01T_ref[...] * inv_e

    blk = inc_ref[...]
    nd = jnp.sum(blk, axis=1, keepdims=True)
    agg = jax.lax.dot_general(blk, msc_ref[...], (((1,), (1,)), ((), ())),
                              preferred_element_type=jnp.float32)
    xc = (1.0 - _ALPHA) * (agg * jax.lax.rsqrt(nd)) + _ALPHA * x_ref[...]
    out_ref[...] = (1.0 - _BETA) * xc + _BETA * jax.lax.dot_general(
        xc, wT_ref[...], (((1,), (0,)), ((), ())),
        preferred_element_type=jnp.float32)


def kernel(x_0, incidence_1, W):
    N, D = x_0.shape
    E = incidence_1.shape[1]
    G = N // _NB

    m01T, colsum, wsum = pl.pallas_call(
        _pass1,
        grid=(G,),
        in_specs=[
            pl.BlockSpec((_NB, D), lambda i: (i, 0)),
            pl.BlockSpec((_NB, E), lambda i: (i, 0)),
        ],
        out_specs=[
            pl.BlockSpec((D, E), lambda i: (0, 0)),
            pl.BlockSpec((1, E), lambda i: (0, 0)),
            pl.BlockSpec((1, E), lambda i: (0, 0)),
        ],
        out_shape=[
            jax.ShapeDtypeStruct((D, E), jnp.float32),
            jax.ShapeDtypeStruct((1, E), jnp.float32),
            jax.ShapeDtypeStruct((1, E), jnp.float32),
        ],
    )(x_0, incidence_1)

    return pl.pallas_call(
        _pass2,
        grid=(G,),
        in_specs=[
            pl.BlockSpec((_NB, E), lambda i: (i, 0)),
            pl.BlockSpec((_NB, D), lambda i: (i, 0)),
            pl.BlockSpec((D, E), lambda i: (0, 0)),
            pl.BlockSpec((1, E), lambda i: (0, 0)),
            pl.BlockSpec((1, E), lambda i: (0, 0)),
            pl.BlockSpec((D, D), lambda i: (0, 0)),
        ],
        out_specs=pl.BlockSpec((_NB, D), lambda i: (i, 0)),
        out_shape=jax.ShapeDtypeStruct((N, D), jnp.float32),
        scratch_shapes=[pltpu.VMEM((D, E), jnp.float32)],
    )(incidence_1, x_0, m01T, colsum, wsum, W.T)


# speedup vs baseline: 1.7230x; 1.0088x over previous
import jax
import jax.numpy as jnp
from jax.experimental import pallas as pl
from jax.experimental.pallas import tpu as pltpu

_ALPHA = 0.1
_BETA = 0.5
_NB = 400


def _pass1(x_ref, inc_ref, m01T_ref, colsum_ref, wsum_ref):
    i = pl.program_id(0)
    blk = inc_ref[...]
    xT = jnp.transpose(x_ref[...])
    m = jax.lax.dot_general(xT.astype(jnp.bfloat16), blk.astype(jnp.bfloat16),
                            (((1,), (0,)), ((), ())),
                            preferred_element_type=jnp.float32)
    nd = jnp.sum(blk, axis=1, keepdims=True)
    cs = jnp.sum(blk, axis=0, keepdims=True)
    ws = jnp.sum(blk * nd, axis=0, keepdims=True)

    @pl.when(i == 0)
    def _init():
        m01T_ref[...] = m
        colsum_ref[...] = cs
        wsum_ref[...] = ws

    @pl.when(i != 0)
    def _acc():
        m01T_ref[...] += m
        colsum_ref[...] += cs
        wsum_ref[...] += ws


def _pass2(inc_ref, x_ref, m01T_ref, colsum_ref, wsum_ref, wT_ref,
           out_ref, msc_ref):
    i = pl.program_id(0)

    @pl.when(i == 0)
    def _prep():
        inv_e = jax.lax.rsqrt(wsum_ref[...] / colsum_ref[...])
        msc_ref[...] = (m01T_ref[...] * inv_e).astype(jnp.bfloat16)

    blk = inc_ref[...]
    nd = jnp.sum(blk, axis=1, keepdims=True)
    agg = jax.lax.dot_general(blk.astype(jnp.bfloat16), msc_ref[...],
                              (((1,), (1,)), ((), ())),
                              preferred_element_type=jnp.float32)
    xc = (1.0 - _ALPHA) * (agg * jax.lax.rsqrt(nd)) + _ALPHA * x_ref[...]
    out_ref[...] = (1.0 - _BETA) * xc + _BETA * jax.lax.dot_general(
        xc.astype(jnp.bfloat16), wT_ref[...].astype(jnp.bfloat16),
        (((1,), (0,)), ((), ())),
        preferred_element_type=jnp.float32)


def kernel(x_0, incidence_1, W):
    N, D = x_0.shape
    E = incidence_1.shape[1]
    G = N // _NB

    m01T, colsum, wsum = pl.pallas_call(
        _pass1,
        grid=(G,),
        in_specs=[
            pl.BlockSpec((_NB, D), lambda i: (i, 0)),
            pl.BlockSpec((_NB, E), lambda i: (i, 0)),
        ],
        out_specs=[
            pl.BlockSpec((D, E), lambda i: (0, 0)),
            pl.BlockSpec((1, E), lambda i: (0, 0)),
            pl.BlockSpec((1, E), lambda i: (0, 0)),
        ],
        out_shape=[
            jax.ShapeDtypeStruct((D, E), jnp.float32),
            jax.ShapeDtypeStruct((1, E), jnp.float32),
            jax.ShapeDtypeStruct((1, E), jnp.float32),
        ],
    )(x_0, incidence_1)

    return pl.pallas_call(
        _pass2,
        grid=(G,),
        in_specs=[
            pl.BlockSpec((_NB, E), lambda i: (i, 0)),
            pl.BlockSpec((_NB, D), lambda i: (i, 0)),
            pl.BlockSpec((D, E), lambda i: (0, 0)),
            pl.BlockSpec((1, E), lambda i: (0, 0)),
            pl.BlockSpec((1, E), lambda i: (0, 0)),
            pl.BlockSpec((D, D), lambda i: (0, 0)),
        ],
        out_specs=pl.BlockSpec((_NB, D), lambda i: (i, 0)),
        out_shape=jax.ShapeDtypeStruct((N, D), jnp.float32),
        scratch_shapes=[pltpu.VMEM((D, E), jnp.bfloat16)],
    )(incidence_1, x_0, m01T, colsum, wsum, W.T)


# speedup vs baseline: 1.8725x; 1.0868x over previous
import jax
import jax.numpy as jnp
from jax.experimental import pallas as pl
from jax.experimental.pallas import tpu as pltpu

_ALPHA = 0.1
_BETA = 0.5
_NB = 1000


def _pass1(x_ref, inc_ref, m01T_ref, colsum_ref, wsum_ref):
    i = pl.program_id(0)
    blk = inc_ref[...]
    xT = jnp.transpose(x_ref[...])
    m = jax.lax.dot_general(xT.astype(jnp.bfloat16), blk.astype(jnp.bfloat16),
                            (((1,), (0,)), ((), ())),
                            preferred_element_type=jnp.float32)
    nd = jnp.sum(blk, axis=1, keepdims=True)
    cs = jnp.sum(blk, axis=0, keepdims=True)
    ws = jnp.sum(blk * nd, axis=0, keepdims=True)

    @pl.when(i == 0)
    def _init():
        m01T_ref[...] = m
        colsum_ref[...] = cs
        wsum_ref[...] = ws

    @pl.when(i != 0)
    def _acc():
        m01T_ref[...] += m
        colsum_ref[...] += cs
        wsum_ref[...] += ws


def _pass2(inc_ref, x_ref, m01T_ref, colsum_ref, wsum_ref, wT_ref,
           out_ref, msc_ref):
    i = pl.program_id(0)

    @pl.when(i == 0)
    def _prep():
        inv_e = jax.lax.rsqrt(wsum_ref[...] / colsum_ref[...])
        msc_ref[...] = (m01T_ref[...] * inv_e).astype(jnp.bfloat16)

    blk = inc_ref[...]
    nd = jnp.sum(blk, axis=1, keepdims=True)
    agg = jax.lax.dot_general(blk.astype(jnp.bfloat16), msc_ref[...],
                              (((1,), (1,)), ((), ())),
                              preferred_element_type=jnp.float32)
    xc = (1.0 - _ALPHA) * (agg * jax.lax.rsqrt(nd)) + _ALPHA * x_ref[...]
    out_ref[...] = (1.0 - _BETA) * xc + _BETA * jax.lax.dot_general(
        xc.astype(jnp.bfloat16), wT_ref[...].astype(jnp.bfloat16),
        (((1,), (0,)), ((), ())),
        preferred_element_type=jnp.float32)


def kernel(x_0, incidence_1, W):
    N, D = x_0.shape
    E = incidence_1.shape[1]
    G = N // _NB

    m01T, colsum, wsum = pl.pallas_call(
        _pass1,
        grid=(G,),
        in_specs=[
            pl.BlockSpec((_NB, D), lambda i: (i, 0)),
            pl.BlockSpec((_NB, E), lambda i: (i, 0)),
        ],
        out_specs=[
            pl.BlockSpec((D, E), lambda i: (0, 0)),
            pl.BlockSpec((1, E), lambda i: (0, 0)),
            pl.BlockSpec((1, E), lambda i: (0, 0)),
        ],
        out_shape=[
            jax.ShapeDtypeStruct((D, E), jnp.float32),
            jax.ShapeDtypeStruct((1, E), jnp.float32),
            jax.ShapeDtypeStruct((1, E), jnp.float32),
        ],
    )(x_0, incidence_1)

    return pl.pallas_call(
        _pass2,
        grid=(G,),
        in_specs=[
            pl.BlockSpec((_NB, E), lambda i: (i, 0)),
            pl.BlockSpec((_NB, D), lambda i: (i, 0)),
            pl.BlockSpec((D, E), lambda i: (0, 0)),
            pl.BlockSpec((1, E), lambda i: (0, 0)),
            pl.BlockSpec((1, E), lambda i: (0, 0)),
            pl.BlockSpec((D, D), lambda i: (0, 0)),
        ],
        out_specs=pl.BlockSpec((_NB, D), lambda i: (i, 0)),
        out_shape=jax.ShapeDtypeStruct((N, D), jnp.float32),
        scratch_shapes=[pltpu.VMEM((D, E), jnp.bfloat16)],
    )(incidence_1, x_0, m01T, colsum, wsum, W.T)
